# baseline (speedup 1.0000x reference)
import functools

import jax
import jax.numpy as jnp
from jax import lax
from jax.experimental import pallas as pl
from jax.experimental.pallas import tpu as pltpu
from jax.experimental.pallas import tpu_sc as plsc

N = 10000
N_PAD = 10240
E = 320000
D_IN = 128
D_OUT = 128

NC = 2
NS = 16
NT = NC * NS
LANES = 16

CHUNK = 64
E_PAD = 327680
EDGES_PER_TILE = E_PAD // NT
NCHUNKS = EDGES_PER_TILE // CHUNK
NBUF = 4
IBLK = 16
NBLOCKS = NCHUNKS // IBLK
RING = 2 * IBLK
ROWS_PER_TILE = N_PAD // NS
WBLK = 128

MM_BLK = 1024


def _matmul_body(x_ref, w_ref, o_ref):
    o_ref[...] = jnp.dot(x_ref[...], w_ref[...], preferred_element_type=jnp.float32)


def _tc_matmul(x, W):
    return pl.pallas_call(
        _matmul_body,
        grid=(N_PAD // MM_BLK,),
        in_specs=[
            pl.BlockSpec((MM_BLK, D_IN), lambda i: (i, 0)),
            pl.BlockSpec((D_IN, D_OUT), lambda i: (0, 0)),
        ],
        out_specs=pl.BlockSpec((MM_BLK, D_OUT), lambda i: (i, 0)),
        out_shape=jax.ShapeDtypeStruct((N_PAD, D_OUT), jnp.float32),
    )(x, W)


def _add_body(a_ref, b_ref, o_ref):
    o_ref[...] = a_ref[...] + b_ref[...]


def _tc_add(a, b):
    return pl.pallas_call(
        _add_body,
        grid=(N_PAD // MM_BLK,),
        in_specs=[
            pl.BlockSpec((MM_BLK, D_OUT), lambda i: (i, 0)),
            pl.BlockSpec((MM_BLK, D_OUT), lambda i: (i, 0)),
        ],
        out_specs=pl.BlockSpec((MM_BLK, D_OUT), lambda i: (i, 0)),
        out_shape=jax.ShapeDtypeStruct((N_PAD, D_OUT), jnp.float32),
    )(a, b)


def _bcast_lane(v, i):
    idx = jnp.full((LANES,), i, dtype=jnp.int32)
    return lax.gather(
        v,
        idx[:, None],
        dimension_numbers=lax.GatherDimensionNumbers(
            offset_dims=(), collapsed_slice_dims=(0,), start_index_map=(0,)
        ),
        slice_sizes=(1,),
        mode=lax.GatherScatterMode.PROMISE_IN_BOUNDS,
    )


def _sc_spmm_body(
    ps, rows_hbm, cols_hbm, vals_hbm, out0, out1,
    cols_ring, rows_ring, vals_ring,
    cc0, cc1, cc2, cc3, rc0, rc1, rc2, rc3, b0, b1, b2, b3, acc,
    bsemC, bsemR, bsemV,
    gs0, gs1, gs2, gs3, ss0, ss1, ss2, ss3, wsem,
):
    cc = lax.axis_index("c")
    s = lax.axis_index("s")
    tid = cc * NS + s

    bufs = [b0, b1, b2, b3]
    colsc = [cc0, cc1, cc2, cc3]
    rowsc = [rc0, rc1, rc2, rc3]
    gsem = [gs0, gs1, gs2, gs3]
    ssem = [ss0, ss1, ss2, ss3]

    zero16 = jnp.zeros((LANES,), jnp.float32)

    def zrow(i, carry):
        for j in range(D_OUT // LANES):
            b0[i, pl.ds(j * LANES, LANES)] = zero16
        return carry

    lax.fori_loop(0, CHUNK, zrow, 0)
    row0 = s * ROWS_PER_TILE
    for b in range(ROWS_PER_TILE // CHUNK):
        pltpu.async_copy(b0, acc.at[pl.ds(row0 + b * CHUNK, CHUNK)], wsem)
    for b in range(ROWS_PER_TILE // CHUNK):
        pltpu.make_async_copy(b0, acc.at[pl.ds(row0, CHUNK)], wsem).wait()
    plsc.subcore_barrier()

    clamp = NCHUNKS - 1

    def blk_load(nb):
        nb = jnp.minimum(nb, NBLOCKS - 1)
        slot = (nb & 1) * IBLK
        pltpu.async_copy(cols_hbm.at[tid, pl.ds(nb * IBLK, IBLK)],
                         cols_ring.at[pl.ds(slot, IBLK)], bsemC)
        pltpu.async_copy(rows_hbm.at[tid, pl.ds(nb * IBLK, IBLK)],
                         rows_ring.at[pl.ds(slot, IBLK)], bsemR)
        pltpu.async_copy(vals_hbm.at[tid, pl.ds(nb * IBLK, IBLK)],
                         vals_ring.at[pl.ds(slot, IBLK)], bsemV)

    def blk_wait():
        pltpu.make_async_copy(cols_hbm.at[0, pl.ds(0, IBLK)],
                              cols_ring.at[pl.ds(0, IBLK)], bsemC).wait()
        pltpu.make_async_copy(rows_hbm.at[0, pl.ds(0, IBLK)],
                              rows_ring.at[pl.ds(0, IBLK)], bsemR).wait()
        pltpu.make_async_copy(vals_hbm.at[0, pl.ds(0, IBLK)],
                              vals_ring.at[pl.ds(0, IBLK)], bsemV).wait()

    def copy_cols(j, m):
        jj = j & (RING - 1)
        for g in range(CHUNK // LANES):
            sl = pl.ds(g * LANES, LANES)
            colsc[m][sl] = cols_ring[jj, sl]

    def copy_rows(j, m):
        jj = j & (RING - 1)
        for g in range(CHUNK // LANES):
            sl = pl.ds(g * LANES, LANES)
            rowsc[m][sl] = rows_ring[jj, sl]

    def gather(j, m):
        pltpu.async_copy(ps.at[colsc[m]], bufs[m], gsem[m])

    def gather_wait(m):
        pltpu.make_async_copy(ps.at[pl.ds(0, CHUNK)], bufs[m], gsem[m]).wait()

    def scatter(j, m):
        del j
        pltpu.async_copy(bufs[m], acc.at[rowsc[m]], ssem[m], add=True)

    def scatter_wait(m):
        pltpu.make_async_copy(bufs[m], acc.at[pl.ds(0, CHUNK)], ssem[m]).wait()

    def scale(j, m):
        buf = bufs[m]
        jj = j & (RING - 1)
        for g in range(CHUNK // LANES):
            vv = vals_ring[jj, pl.ds(g * LANES, LANES)]
            for i in range(LANES):
                e = g * LANES + i
                vb = _bcast_lane(vv, i)
                for jf in range(D_OUT // LANES):
                    sl = pl.ds(jf * LANES, LANES)
                    buf[e, sl] = buf[e, sl] * vb

    blk_load(0)
    blk_wait()
    copy_cols(0, 0)
    copy_cols(1, 1)
    gather(0, 0)
    gather(1, 1)

    for j in (0, 1):
        m = j
        copy_cols(j + 2, j + 2)
        gather(j + 2, j + 2)
        gather_wait(m)
        copy_rows(j, m)
        scale(j, m)
        scatter(j, m)

    def sbody(j, cur, n2):
        scatter_wait(n2)
        copy_cols(j + 2, n2)
        gather(j + 2, n2)
        gather_wait(cur)
        copy_rows(j, cur)
        scale(j, cur)
        scatter(j, cur)

    def body(k, carry):
        j = 4 * k + 2

        @pl.when((k & 3) == 3)
        def _():
            blk_wait()

        @pl.when((k & 3) == 1)
        def _():
            blk_load((j >> 4) + 1)

        sbody(j, 2, 0)
        sbody(j + 1, 3, 1)
        sbody(j + 2, 0, 2)
        sbody(j + 3, 1, 3)
        return carry

    lax.fori_loop(0, (NCHUNKS - 4) // 4, body, 0)

    for j, m in ((NCHUNKS - 2, 2), (NCHUNKS - 1, 3)):
        gather_wait(m)
        copy_rows(j, m)
        scale(j, m)
        scatter(j, m)

    for m in range(NBUF):
        scatter_wait(m)
    blk_wait()
    plsc.subcore_barrier()

    @pl.when(cc == 0)
    def _():
        for b in range(ROWS_PER_TILE // WBLK):
            r = row0 + b * WBLK
            pltpu.async_copy(acc.at[pl.ds(r, WBLK)], out0.at[pl.ds(r, WBLK)], wsem)
        for b in range(ROWS_PER_TILE // WBLK):
            pltpu.make_async_copy(acc.at[pl.ds(row0, WBLK)], out0.at[pl.ds(row0, WBLK)], wsem).wait()

    @pl.when(cc == 1)
    def _():
        for b in range(ROWS_PER_TILE // WBLK):
            r = row0 + b * WBLK
            pltpu.async_copy(acc.at[pl.ds(r, WBLK)], out1.at[pl.ds(r, WBLK)], wsem)
        for b in range(ROWS_PER_TILE // WBLK):
            pltpu.make_async_copy(acc.at[pl.ds(row0, WBLK)], out1.at[pl.ds(row0, WBLK)], wsem).wait()


_sc_spmm = functools.partial(
    pl.kernel,
    mesh=plsc.VectorSubcoreMesh(core_axis_name="c", subcore_axis_name="s"),
    out_type=[
        jax.ShapeDtypeStruct((N_PAD, D_OUT), jnp.float32),
        jax.ShapeDtypeStruct((N_PAD, D_OUT), jnp.float32),
    ],
    scratch_types=(
        [pltpu.VMEM((RING, CHUNK), jnp.int32)]
        + [pltpu.VMEM((RING, CHUNK), jnp.int32)]
        + [pltpu.VMEM((RING, CHUNK), jnp.float32)]
        + [pltpu.VMEM((CHUNK,), jnp.int32) for _ in range(4)]
        + [pltpu.VMEM((CHUNK,), jnp.int32) for _ in range(4)]
        + [pltpu.VMEM((CHUNK, D_OUT), jnp.float32) for _ in range(4)]
        + [pltpu.VMEM_SHARED((N_PAD, D_OUT), jnp.float32)]
        + [pltpu.SemaphoreType.DMA for _ in range(12)]
    ),
)(_sc_spmm_body)


def kernel(x, adj_indices, adj_values, W):
    x_pad = jnp.pad(x, ((0, N_PAD - N), (0, 0)))
    ps = _tc_matmul(x_pad, W)
    rows = jnp.pad(adj_indices[0], (0, E_PAD - E)).reshape(NT, NCHUNKS, CHUNK)
    cols = jnp.pad(adj_indices[1], (0, E_PAD - E)).reshape(NT, NCHUNKS, CHUNK)
    vals = jnp.pad(adj_values, (0, E_PAD - E)).reshape(NT, NCHUNKS, CHUNK)
    p0, p1 = _sc_spmm(ps, rows, cols, vals)
    return _tc_add(p0, p1)[:N]


# speedup vs baseline: 1.0024x; 1.0024x over previous
import functools

import jax
import jax.numpy as jnp
from jax import lax
from jax.experimental import pallas as pl
from jax.experimental.pallas import tpu as pltpu
from jax.experimental.pallas import tpu_sc as plsc

N = 10000
N_PAD = 10240
E = 320000
D_IN = 128
D_OUT = 128

NC = 2
NS = 16
NT = NC * NS
LANES = 16

CHUNK = 64
E_PAD = 327680
EDGES_PER_TILE = E_PAD // NT
NCHUNKS = EDGES_PER_TILE // CHUNK
NBUF = 4
IBLK = 4
NBLOCKS = NCHUNKS // IBLK
RING = 4 * IBLK
ROWS_PER_TILE = N_PAD // NS
WBLK = 128

MM_BLK = 1024


def _matmul_body(x_ref, w_ref, o_ref):
    o_ref[...] = jnp.dot(x_ref[...], w_ref[...], preferred_element_type=jnp.float32)


def _tc_matmul(x, W):
    return pl.pallas_call(
        _matmul_body,
        grid=(N_PAD // MM_BLK,),
        in_specs=[
            pl.BlockSpec((MM_BLK, D_IN), lambda i: (i, 0)),
            pl.BlockSpec((D_IN, D_OUT), lambda i: (0, 0)),
        ],
        out_specs=pl.BlockSpec((MM_BLK, D_OUT), lambda i: (i, 0)),
        out_shape=jax.ShapeDtypeStruct((N_PAD, D_OUT), jnp.float32),
    )(x, W)


def _add_body(a_ref, b_ref, o_ref):
    o_ref[...] = a_ref[...] + b_ref[...]


def _tc_add(a, b):
    return pl.pallas_call(
        _add_body,
        grid=(N_PAD // MM_BLK,),
        in_specs=[
            pl.BlockSpec((MM_BLK, D_OUT), lambda i: (i, 0)),
            pl.BlockSpec((MM_BLK, D_OUT), lambda i: (i, 0)),
        ],
        out_specs=pl.BlockSpec((MM_BLK, D_OUT), lambda i: (i, 0)),
        out_shape=jax.ShapeDtypeStruct((N_PAD, D_OUT), jnp.float32),
    )(a, b)


def _bcast_lane(v, i):
    idx = jnp.full((LANES,), i, dtype=jnp.int32)
    return lax.gather(
        v,
        idx[:, None],
        dimension_numbers=lax.GatherDimensionNumbers(
            offset_dims=(), collapsed_slice_dims=(0,), start_index_map=(0,)
        ),
        slice_sizes=(1,),
        mode=lax.GatherScatterMode.PROMISE_IN_BOUNDS,
    )


def _sc_spmm_body(
    ps, rows_hbm, cols_hbm, vals_hbm, out0, out1,
    cols_ring, rows_ring, vals_ring,
    cc0, cc1, cc2, cc3, rc0, rc1, rc2, rc3, b0, b1, b2, b3, acc,
    bsemC, bsemR, bsemV,
    gs0, gs1, gs2, gs3, ss0, ss1, ss2, ss3, wsem,
):
    cc = lax.axis_index("c")
    s = lax.axis_index("s")
    tid = cc * NS + s

    bufs = [b0, b1, b2, b3]
    colsc = [cc0, cc1, cc2, cc3]
    rowsc = [rc0, rc1, rc2, rc3]
    gsem = [gs0, gs1, gs2, gs3]
    ssem = [ss0, ss1, ss2, ss3]

    zero16 = jnp.zeros((LANES,), jnp.float32)

    def zrow(i, carry):
        for j in range(D_OUT // LANES):
            b0[i, pl.ds(j * LANES, LANES)] = zero16
        return carry

    lax.fori_loop(0, CHUNK, zrow, 0)
    row0 = s * ROWS_PER_TILE
    for b in range(ROWS_PER_TILE // CHUNK):
        pltpu.async_copy(b0, acc.at[pl.ds(row0 + b * CHUNK, CHUNK)], wsem)
    for b in range(ROWS_PER_TILE // CHUNK):
        pltpu.make_async_copy(b0, acc.at[pl.ds(row0, CHUNK)], wsem).wait()
    plsc.subcore_barrier()

    clamp = NCHUNKS - 1

    def blk_load(nb):
        nb = jnp.minimum(nb, NBLOCKS - 1)
        slot = (nb & 3) * IBLK
        pltpu.async_copy(cols_hbm.at[tid, pl.ds(nb * IBLK, IBLK)],
                         cols_ring.at[pl.ds(slot, IBLK)], bsemC)
        pltpu.async_copy(rows_hbm.at[tid, pl.ds(nb * IBLK, IBLK)],
                         rows_ring.at[pl.ds(slot, IBLK)], bsemR)
        pltpu.async_copy(vals_hbm.at[tid, pl.ds(nb * IBLK, IBLK)],
                         vals_ring.at[pl.ds(slot, IBLK)], bsemV)

    def blk_wait():
        pltpu.make_async_copy(cols_hbm.at[0, pl.ds(0, IBLK)],
                              cols_ring.at[pl.ds(0, IBLK)], bsemC).wait()
        pltpu.make_async_copy(rows_hbm.at[0, pl.ds(0, IBLK)],
                              rows_ring.at[pl.ds(0, IBLK)], bsemR).wait()
        pltpu.make_async_copy(vals_hbm.at[0, pl.ds(0, IBLK)],
                              vals_ring.at[pl.ds(0, IBLK)], bsemV).wait()

    def copy_cols(j, m):
        jj = j & (RING - 1)
        for g in range(CHUNK // LANES):
            sl = pl.ds(g * LANES, LANES)
            colsc[m][sl] = cols_ring[jj, sl]

    def copy_rows(j, m):
        jj = j & (RING - 1)
        for g in range(CHUNK // LANES):
            sl = pl.ds(g * LANES, LANES)
            rowsc[m][sl] = rows_ring[jj, sl]

    def gather(j, m):
        pltpu.async_copy(ps.at[colsc[m]], bufs[m], gsem[m])

    def gather_wait(m):
        pltpu.make_async_copy(ps.at[pl.ds(0, CHUNK)], bufs[m], gsem[m]).wait()

    def scatter(j, m):
        del j
        pltpu.async_copy(bufs[m], acc.at[rowsc[m]], ssem[m], add=True)

    def scatter_wait(m):
        pltpu.make_async_copy(bufs[m], acc.at[pl.ds(0, CHUNK)], ssem[m]).wait()

    def scale(j, m):
        buf = bufs[m]
        jj = j & (RING - 1)
        for g in range(CHUNK // LANES):
            vv = vals_ring[jj, pl.ds(g * LANES, LANES)]
            for i in range(LANES):
                e = g * LANES + i
                vb = _bcast_lane(vv, i)
                for jf in range(D_OUT // LANES):
                    sl = pl.ds(jf * LANES, LANES)
                    buf[e, sl] = buf[e, sl] * vb

    blk_load(0)
    blk_load(1)
    blk_wait()
    copy_cols(0, 0)
    copy_cols(1, 1)
    gather(0, 0)
    gather(1, 1)

    for j in (0, 1):
        m = j
        copy_cols(j + 2, j + 2)
        gather(j + 2, j + 2)
        gather_wait(m)
        copy_rows(j, m)
        scale(j, m)
        scatter(j, m)

    def sbody(j, cur, n2):
        scatter_wait(n2)
        copy_cols(j + 2, n2)
        gather(j + 2, n2)
        gather_wait(cur)
        copy_rows(j, cur)
        scale(j, cur)
        scatter(j, cur)

    def body(k, carry):
        j = 4 * k + 2
        blk_wait()
        blk_load(k + 2)
        sbody(j, 2, 0)
        sbody(j + 1, 3, 1)
        sbody(j + 2, 0, 2)
        sbody(j + 3, 1, 3)
        return carry

    lax.fori_loop(0, (NCHUNKS - 4) // 4, body, 0)

    for j, m in ((NCHUNKS - 2, 2), (NCHUNKS - 1, 3)):
        gather_wait(m)
        copy_rows(j, m)
        scale(j, m)
        scatter(j, m)

    for m in range(NBUF):
        scatter_wait(m)
    blk_wait()
    plsc.subcore_barrier()

    @pl.when(cc == 0)
    def _():
        for b in range(ROWS_PER_TILE // WBLK):
            r = row0 + b * WBLK
            pltpu.async_copy(acc.at[pl.ds(r, WBLK)], out0.at[pl.ds(r, WBLK)], wsem)
        for b in range(ROWS_PER_TILE // WBLK):
            pltpu.make_async_copy(acc.at[pl.ds(row0, WBLK)], out0.at[pl.ds(row0, WBLK)], wsem).wait()

    @pl.when(cc == 1)
    def _():
        for b in range(ROWS_PER_TILE // WBLK):
            r = row0 + b * WBLK
            pltpu.async_copy(acc.at[pl.ds(r, WBLK)], out1.at[pl.ds(r, WBLK)], wsem)
        for b in range(ROWS_PER_TILE // WBLK):
            pltpu.make_async_copy(acc.at[pl.ds(row0, WBLK)], out1.at[pl.ds(row0, WBLK)], wsem).wait()


_sc_spmm = functools.partial(
    pl.kernel,
    mesh=plsc.VectorSubcoreMesh(core_axis_name="c", subcore_axis_name="s"),
    out_type=[
        jax.ShapeDtypeStruct((N_PAD, D_OUT), jnp.float32),
        jax.ShapeDtypeStruct((N_PAD, D_OUT), jnp.float32),
    ],
    scratch_types=(
        [pltpu.VMEM((RING, CHUNK), jnp.int32)]
        + [pltpu.VMEM((RING, CHUNK), jnp.int32)]
        + [pltpu.VMEM((RING, CHUNK), jnp.float32)]
        + [pltpu.VMEM((CHUNK,), jnp.int32) for _ in range(4)]
        + [pltpu.VMEM((CHUNK,), jnp.int32) for _ in range(4)]
        + [pltpu.VMEM((CHUNK, D_OUT), jnp.float32) for _ in range(4)]
        + [pltpu.VMEM_SHARED((N_PAD, D_OUT), jnp.float32)]
        + [pltpu.SemaphoreType.DMA for _ in range(12)]
    ),
)(_sc_spmm_body)


def kernel(x, adj_indices, adj_values, W):
    x_pad = jnp.pad(x, ((0, N_PAD - N), (0, 0)))
    ps = _tc_matmul(x_pad, W)
    rows = jnp.pad(adj_indices[0], (0, E_PAD - E)).reshape(NT, NCHUNKS, CHUNK)
    cols = jnp.pad(adj_indices[1], (0, E_PAD - E)).reshape(NT, NCHUNKS, CHUNK)
    vals = jnp.pad(adj_values, (0, E_PAD - E)).reshape(NT, NCHUNKS, CHUNK)
    p0, p1 = _sc_spmm(ps, rows, cols, vals)
    return _tc_add(p0, p1)[:N]
